# Initial kernel scaffold; baseline (speedup 1.0000x reference)
import jax
import jax.numpy as jnp
from jax.experimental import pallas as pl


def kernel(coord, feat, offset, params):
    raise NotImplementedError("write your pallas kernel here")


# speedup vs baseline: 4.3792x; 4.3792x over previous
import functools

import jax
import jax.numpy as jnp
from jax.experimental import pallas as pl
from jax.experimental.pallas import tpu as pltpu

_N = 8192
_B = 8
_PE_G = 6
_PE_K = 8
_ENC_CH = (96, 192, 384, 512)
_ENC_G = (12, 24, 48, 64)
_ENC_K = (16, 16, 16, 16)
_GRIDS = (0.06, 0.15, 0.375, 0.9375)


def _linear_kern(x_ref, w_ref, b_ref, o_ref, *, relu):
    y = jnp.dot(x_ref[...], w_ref[...], preferred_element_type=jnp.float32)
    y = y + b_ref[...]
    if relu:
        y = jnp.maximum(y, 0.0)
    o_ref[...] = y


def _plinear(x, w, b, relu=False, tile=512):
    n, din = x.shape
    dout = w.shape[1]
    if n < tile:
        tile = n
    return pl.pallas_call(
        functools.partial(_linear_kern, relu=relu),
        grid=(n // tile,),
        in_specs=[
            pl.BlockSpec((tile, din), lambda i: (i, 0)),
            pl.BlockSpec((din, dout), lambda i: (0, 0)),
            pl.BlockSpec((1, dout), lambda i: (0, 0)),
        ],
        out_specs=pl.BlockSpec((tile, dout), lambda i: (i, 0)),
        out_shape=jax.ShapeDtypeStruct((n, dout), jnp.float32),
    )(x, w, b.reshape(1, -1))


def _knn_kern(s_ref, cnt_ref, c_ref, ct3_ref, o_ref, *, K, RT, NCH):
    b = pl.program_id(0)
    t = pl.program_id(1)

    @pl.when((b == 0) & (t == 0))
    def _init():
        o_ref[...] = jnp.zeros(o_ref.shape, o_ref.dtype)

    start = s_ref[b]
    cnt = cnt_ref[b]
    row0 = start + t * RT
    rows = c_ref[pl.ds(row0, RT), :]
    chunk = start // 1024
    chunk1 = jnp.minimum(chunk + 1, NCH - 1)
    d_halves = []
    jg_halves = []
    for which, cidx in enumerate((chunk, chunk1)):
        cx = ct3_ref[0, pl.ds(cidx, 1), :]
        cy = ct3_ref[1, pl.ds(cidx, 1), :]
        cz = ct3_ref[2, pl.ds(cidx, 1), :]
        dx = rows[:, 0:1] - cx
        dy = rows[:, 1:2] - cy
        dz = rows[:, 2:3] - cz
        d = (dx * dx + dy * dy) + dz * dz
        jg = jax.lax.broadcasted_iota(jnp.int32, (RT, 1024), 1) + cidx * 1024
        mask = (jg >= start) & (jg < start + cnt)
        if which == 1:
            mask = mask & (chunk1 != chunk)
        d = jnp.where(mask, d, jnp.inf)
        d_halves.append(d)
        jg_halves.append(jg)
    d = jnp.concatenate(d_halves, axis=1)
    jg = jnp.concatenate(jg_halves, axis=1)
    cols = []
    for _ in range(K):
        m = jnp.min(d, axis=1, keepdims=True)
        ismin = d == m
        fidx = jnp.min(jnp.where(ismin, jg, _N), axis=1, keepdims=True)
        cols.append(fidx)
        d = jnp.where(jg == fidx, jnp.inf, d)
    out = jnp.concatenate(cols, axis=1).astype(jnp.int32)
    o_ref[pl.ds(row0, RT), :] = out


def _knn_pallas(c, starts, cnts, bidv, validv, bcount, k):
    n = c.shape[0]
    RT = 128
    ct3 = c.T.reshape(3, n // 1024, 1024)
    nb = pl.pallas_call(
        functools.partial(_knn_kern, K=k, RT=RT, NCH=n // 1024),
        grid=(_B, 1024 // RT),
        in_specs=[
            pl.BlockSpec(memory_space=pltpu.MemorySpace.SMEM),
            pl.BlockSpec(memory_space=pltpu.MemorySpace.SMEM),
            pl.BlockSpec((n, 3), lambda b, t: (0, 0)),
            pl.BlockSpec((3, n // 1024, 1024), lambda b, t: (0, 0, 0)),
        ],
        out_specs=pl.BlockSpec((n, k), lambda b, t: (0, 0)),
        out_shape=jax.ShapeDtypeStruct((n, k), jnp.int32),
    )(starts, cnts, c, ct3)
    vcnt = bcount[jnp.where(validv, bidv, 0)]
    col = jnp.arange(k)[None, :]
    last = jnp.take_along_axis(nb, jnp.clip(vcnt - 1, 0, k - 1)[:, None], axis=1)
    return jnp.where(col < vcnt[:, None], nb, last)


def _gva_tail_kern(q_ref, feat_ref, kn_ref, vn_ref, pos_ref,
                   pe1_ref, pe1b_ref, pe2_ref, pe2b_ref,
                   we1_ref, we1b_ref, we2_ref, we2b_ref,
                   gmat_ref, proj_ref, projb_ref, o_ref, *, K, G):
    TN, ch = q_ref.shape
    TNK = TN * K
    pos = pos_ref[...]
    h = jnp.dot(pos, pe1_ref[...], preferred_element_type=jnp.float32)
    h = jnp.maximum(h + pe1b_ref[...], 0.0)
    pe = jnp.dot(h, pe2_ref[...], preferred_element_type=jnp.float32)
    pe = pe + pe2b_ref[...]
    rel = (q_ref[...].reshape(TN, 1, ch)
           - kn_ref[...].reshape(TN, K, ch)
           + pe.reshape(TN, K, ch)).reshape(TNK, ch)
    t = jnp.dot(rel, we1_ref[...], preferred_element_type=jnp.float32)
    t = jnp.maximum(t + we1b_ref[...], 0.0)
    w = jnp.dot(t, we2_ref[...], preferred_element_type=jnp.float32)
    w = (w + we2b_ref[...]).reshape(TN, K, G)
    w = w - jnp.max(w, axis=1, keepdims=True)
    e = jnp.exp(w)
    w = e / jnp.sum(e, axis=1, keepdims=True)
    wfull = jnp.dot(w.reshape(TNK, G), gmat_ref[...],
                    preferred_element_type=jnp.float32)
    val = vn_ref[...] + pe
    out = jnp.sum((wfull * val).reshape(TN, K, ch), axis=1)
    y = jnp.dot(out, proj_ref[...], preferred_element_type=jnp.float32)
    y = y + projb_ref[...]
    o_ref[...] = feat_ref[...] + jnp.maximum(y, 0.0)


def _gva(p, feat, coord, nbr, g):
    n, ch = feat.shape
    k = nbr.shape[1]
    wqkv = jnp.concatenate([p["q"]["w"], p["k"]["w"], p["v"]["w"]], axis=1)
    bqkv = jnp.concatenate([p["q"]["b"], p["k"]["b"], p["v"]["b"]])
    qkv = _plinear(feat, wqkv, bqkv)
    q = qkv[:, :ch]
    flat = nbr.reshape(-1)
    kn = qkv[:, ch:2 * ch][flat]
    vn = qkv[:, 2 * ch:][flat]
    pos = (coord[flat].reshape(n, k, 3) - coord[:, None, :]).reshape(n * k, 3)
    gmat = (jnp.arange(ch)[None, :] // (ch // g)
            == jnp.arange(g)[:, None]).astype(jnp.float32)
    TN = 128
    grid = (n // TN,)
    full = lambda r, c: pl.BlockSpec((r, c), lambda i: (0, 0))
    row = lambda r, c: pl.BlockSpec((r, c), lambda i: (i, 0))
    return pl.pallas_call(
        functools.partial(_gva_tail_kern, K=k, G=g),
        grid=grid,
        in_specs=[
            row(TN, ch), row(TN, ch), row(TN * k, ch), row(TN * k, ch),
            row(TN * k, 3),
            full(3, ch), full(1, ch), full(ch, ch), full(1, ch),
            full(ch, ch), full(1, ch), full(ch, g), full(1, g),
            full(g, ch), full(ch, ch), full(1, ch),
        ],
        out_specs=row(TN, ch),
        out_shape=jax.ShapeDtypeStruct((n, ch), jnp.float32),
    )(q, feat, kn, vn, pos,
      p["pe1"]["w"], p["pe1"]["b"].reshape(1, -1),
      p["pe2"]["w"], p["pe2"]["b"].reshape(1, -1),
      p["we1"]["w"], p["we1"]["b"].reshape(1, -1),
      p["we2"]["w"], p["we2"]["b"].reshape(1, -1),
      gmat, p["proj"]["w"], p["proj"]["b"].reshape(1, -1))


def _grid_cluster(c, bidv, validv, grid):
    n = c.shape[0]
    bid_safe = jnp.where(validv, bidv, 0)
    bmin = jax.ops.segment_min(jnp.where(validv[:, None], c, jnp.inf),
                               bid_safe, num_segments=_B)
    v = jnp.floor((c - bmin[bid_safe]) / grid).astype(jnp.int32)
    keyv = v[:, 0] * 1000000 + v[:, 1] * 1000 + v[:, 2]
    bkey = jnp.where(validv, bidv, _B)
    order = jnp.lexsort((keyv, bkey))
    sk = keyv[order]
    sb = bkey[order]
    flag = jnp.concatenate([
        jnp.zeros(1, jnp.int32),
        ((sk[1:] != sk[:-1]) | (sb[1:] != sb[:-1])).astype(jnp.int32)])
    ids = jnp.cumsum(flag)
    return jnp.zeros(n, jnp.int32).at[order].set(ids)


def _structures(coord, offset):
    n = coord.shape[0]
    bid0 = jnp.searchsorted(offset, jnp.arange(n), side="right").astype(jnp.int32)
    valid0 = jnp.ones(n, dtype=bool)
    starts0 = jnp.concatenate([jnp.zeros(1, jnp.int32), offset[:-1]])
    bcnt0 = offset - starts0
    nbr0 = _knn_pallas(coord, starts0, bcnt0, bid0, valid0, bcnt0, _PE_K)
    stages = []
    cur_c, cur_b, cur_v = coord, bid0, valid0
    for i in range(4):
        clus = _grid_cluster(cur_c, cur_b, cur_v, _GRIDS[i])
        raw_cnt = jax.ops.segment_sum(cur_v.astype(jnp.float32), clus,
                                      num_segments=n)
        seg_valid = raw_cnt > 0
        cnt = jnp.where(seg_valid, raw_cnt, 1.0)
        pooled_c = jax.ops.segment_sum(jnp.where(cur_v[:, None], cur_c, 0.0),
                                       clus, num_segments=n) / cnt[:, None]
        seg_bid = jax.ops.segment_max(jnp.where(cur_v, cur_b, -1), clus,
                                      num_segments=n)
        seg_bid = jnp.where(seg_valid, seg_bid, 0)
        seg_bcnt = jax.ops.segment_sum(seg_valid.astype(jnp.int32), seg_bid,
                                       num_segments=_B)
        seg_starts = (jnp.cumsum(seg_bcnt) - seg_bcnt).astype(jnp.int32)
        nbr = _knn_pallas(pooled_c, seg_starts, seg_bcnt, seg_bid, seg_valid,
                          seg_bcnt, _ENC_K[i])
        stages.append((clus, n, cnt, nbr, seg_valid))
        cur_c, cur_b, cur_v = pooled_c, seg_bid, seg_valid
    bcnt = jax.ops.segment_sum(cur_v.astype(jnp.float32), cur_b,
                               num_segments=_B)
    return nbr0, stages, cur_b, bcnt


def kernel(coord, feat, offset, params):
    nbr0, stages, bid, bcnt = _structures(coord, offset)
    x = coord
    f = _plinear(feat, params["embed"]["w"], params["embed"]["b"])
    f = _gva(params["pe_block"], f, x, nbr0, _PE_G)
    for i in range(4):
        f = _plinear(f, params["down"][i]["w"], params["down"][i]["b"],
                     relu=True)
        clus, nseg, cnt, nbr, seg_valid = stages[i]
        px = jax.ops.segment_sum(x, clus, num_segments=nseg) / cnt[:, None]
        pf = jax.ops.segment_max(f, clus, num_segments=nseg)
        f = _gva(params["blocks"][i], pf, px, nbr, _ENC_G[i])
        x = px
    f = jnp.where(stages[-1][4][:, None], f, 0.0)
    pooled = jax.ops.segment_sum(f, bid, num_segments=bcnt.shape[0]) / bcnt[:, None]
    h = _plinear(pooled, params["head1"]["w"], params["head1"]["b"], relu=True)
    h = _plinear(h, params["head2"]["w"], params["head2"]["b"], relu=True)
    return _plinear(h, params["head3"]["w"], params["head3"]["b"])


# speedup vs baseline: 9.8958x; 2.2597x over previous
import functools

import jax
import jax.numpy as jnp
from jax.experimental import pallas as pl
from jax.experimental.pallas import tpu as pltpu

_N = 8192
_B = 8
_PE_G = 6
_PE_K = 8
_ENC_CH = (96, 192, 384, 512)
_ENC_G = (12, 24, 48, 64)
_ENC_K = (16, 16, 16, 16)
_GRIDS = (0.06, 0.15, 0.375, 0.9375)


def _linear_kern(x_ref, w_ref, b_ref, o_ref, *, relu):
    y = jnp.dot(x_ref[...], w_ref[...], preferred_element_type=jnp.float32)
    y = y + b_ref[...]
    if relu:
        y = jnp.maximum(y, 0.0)
    o_ref[...] = y


def _plinear(x, w, b, relu=False, tile=512):
    n, din = x.shape
    dout = w.shape[1]
    tile = min(tile, n)
    while n % tile:
        tile //= 2
    return pl.pallas_call(
        functools.partial(_linear_kern, relu=relu),
        grid=(n // tile,),
        in_specs=[
            pl.BlockSpec((tile, din), lambda i: (i, 0)),
            pl.BlockSpec((din, dout), lambda i: (0, 0)),
            pl.BlockSpec((1, dout), lambda i: (0, 0)),
        ],
        out_specs=pl.BlockSpec((tile, dout), lambda i: (i, 0)),
        out_shape=jax.ShapeDtypeStruct((n, dout), jnp.float32),
    )(x, w, b.reshape(1, -1))


def _knn_kern(s_ref, cnt_ref, c_ref, ct3_ref, o_ref, *, K, RT, NCH):
    b = pl.program_id(0)
    t = pl.program_id(1)

    @pl.when((b == 0) & (t == 0))
    def _init():
        o_ref[...] = jnp.zeros(o_ref.shape, o_ref.dtype)

    start = s_ref[b]
    cnt = cnt_ref[b]
    row0 = start + t * RT
    rows = c_ref[pl.ds(row0, RT), :]
    chunk = start // 1024
    chunk1 = jnp.minimum(chunk + 1, NCH - 1)
    d_halves = []
    jg_halves = []
    for which, cidx in enumerate((chunk, chunk1)):
        cx = ct3_ref[0, pl.ds(cidx, 1), :]
        cy = ct3_ref[1, pl.ds(cidx, 1), :]
        cz = ct3_ref[2, pl.ds(cidx, 1), :]
        dx = rows[:, 0:1] - cx
        dy = rows[:, 1:2] - cy
        dz = rows[:, 2:3] - cz
        d = (dx * dx + dy * dy) + dz * dz
        jg = jax.lax.broadcasted_iota(jnp.int32, (RT, 1024), 1) + cidx * 1024
        mask = (jg >= start) & (jg < start + cnt)
        if which == 1:
            mask = mask & (chunk1 != chunk)
        d = jnp.where(mask, d, jnp.inf)
        d_halves.append(d)
        jg_halves.append(jg)
    d = jnp.concatenate(d_halves, axis=1)
    jg = jnp.concatenate(jg_halves, axis=1)
    cols = []
    for _ in range(K):
        m = jnp.min(d, axis=1, keepdims=True)
        ismin = d == m
        fidx = jnp.min(jnp.where(ismin, jg, _N), axis=1, keepdims=True)
        cols.append(fidx)
        d = jnp.where(jg == fidx, jnp.inf, d)
    out = jnp.concatenate(cols, axis=1).astype(jnp.int32)
    o_ref[pl.ds(row0, RT), :] = out


def _knn_pallas(c, starts, cnts, bidv, validv, bcount, k, ttiles=8):
    p = c.shape[0]
    n = max(1024, -(-p // 1024) * 1024)
    if n != p:
        c = jnp.zeros((n, 3), c.dtype).at[:p].set(c)
    RT = 128
    ct3 = c.T.reshape(3, n // 1024, 1024)
    nb = pl.pallas_call(
        functools.partial(_knn_kern, K=k, RT=RT, NCH=n // 1024),
        grid=(_B, ttiles),
        in_specs=[
            pl.BlockSpec(memory_space=pltpu.MemorySpace.SMEM),
            pl.BlockSpec(memory_space=pltpu.MemorySpace.SMEM),
            pl.BlockSpec((n, 3), lambda b, t: (0, 0)),
            pl.BlockSpec((3, n // 1024, 1024), lambda b, t: (0, 0, 0)),
        ],
        out_specs=pl.BlockSpec((n, k), lambda b, t: (0, 0)),
        out_shape=jax.ShapeDtypeStruct((n, k), jnp.int32),
    )(starts, cnts, c, ct3)
    nb = nb[:p]
    vcnt = bcount[jnp.where(validv, bidv, 0)]
    col = jnp.arange(k)[None, :]
    last = jnp.take_along_axis(nb, jnp.clip(vcnt - 1, 0, k - 1)[:, None], axis=1)
    return jnp.where(col < vcnt[:, None], nb, last)


def _gva_tail_kern(q_ref, feat_ref, kn_ref, vn_ref, pos_ref,
                   pe1_ref, pe1b_ref, pe2_ref, pe2b_ref,
                   we1_ref, we1b_ref, we2_ref, we2b_ref,
                   gmat_ref, proj_ref, projb_ref, o_ref, *, K, G):
    TN, ch = q_ref.shape
    TNK = TN * K
    pos = pos_ref[...]
    h = jnp.dot(pos, pe1_ref[...], preferred_element_type=jnp.float32)
    h = jnp.maximum(h + pe1b_ref[...], 0.0)
    pe = jnp.dot(h, pe2_ref[...], preferred_element_type=jnp.float32)
    pe = pe + pe2b_ref[...]
    rel = (q_ref[...].reshape(TN, 1, ch)
           - kn_ref[...].reshape(TN, K, ch)
           + pe.reshape(TN, K, ch)).reshape(TNK, ch)
    t = jnp.dot(rel, we1_ref[...], preferred_element_type=jnp.float32)
    t = jnp.maximum(t + we1b_ref[...], 0.0)
    w = jnp.dot(t, we2_ref[...], preferred_element_type=jnp.float32)
    w = (w + we2b_ref[...]).reshape(TN, K, G)
    w = w - jnp.max(w, axis=1, keepdims=True)
    e = jnp.exp(w)
    w = e / jnp.sum(e, axis=1, keepdims=True)
    wfull = jnp.dot(w.reshape(TNK, G), gmat_ref[...],
                    preferred_element_type=jnp.float32)
    val = vn_ref[...] + pe
    out = jnp.sum((wfull * val).reshape(TN, K, ch), axis=1)
    y = jnp.dot(out, proj_ref[...], preferred_element_type=jnp.float32)
    y = y + projb_ref[...]
    o_ref[...] = feat_ref[...] + jnp.maximum(y, 0.0)


def _gva(p, feat, coord, nbr, g):
    n, ch = feat.shape
    k = nbr.shape[1]
    wqkv = jnp.concatenate([p["q"]["w"], p["k"]["w"], p["v"]["w"]], axis=1)
    bqkv = jnp.concatenate([p["q"]["b"], p["k"]["b"], p["v"]["b"]])
    qkv = _plinear(feat, wqkv, bqkv)
    q = qkv[:, :ch]
    flat = nbr.reshape(-1)
    kn = qkv[:, ch:2 * ch][flat]
    vn = qkv[:, 2 * ch:][flat]
    pos = (coord[flat].reshape(n, k, 3) - coord[:, None, :]).reshape(n * k, 3)
    gmat = (jnp.arange(ch)[None, :] // (ch // g)
            == jnp.arange(g)[:, None]).astype(jnp.float32)
    TN = min(128, n)
    grid = (n // TN,)
    full = lambda r, c: pl.BlockSpec((r, c), lambda i: (0, 0))
    row = lambda r, c: pl.BlockSpec((r, c), lambda i: (i, 0))
    return pl.pallas_call(
        functools.partial(_gva_tail_kern, K=k, G=g),
        grid=grid,
        in_specs=[
            row(TN, ch), row(TN, ch), row(TN * k, ch), row(TN * k, ch),
            row(TN * k, 3),
            full(3, ch), full(1, ch), full(ch, ch), full(1, ch),
            full(ch, ch), full(1, ch), full(ch, g), full(1, g),
            full(g, ch), full(ch, ch), full(1, ch),
        ],
        out_specs=row(TN, ch),
        out_shape=jax.ShapeDtypeStruct((n, ch), jnp.float32),
    )(q, feat, kn, vn, pos,
      p["pe1"]["w"], p["pe1"]["b"].reshape(1, -1),
      p["pe2"]["w"], p["pe2"]["b"].reshape(1, -1),
      p["we1"]["w"], p["we1"]["b"].reshape(1, -1),
      p["we2"]["w"], p["we2"]["b"].reshape(1, -1),
      gmat, p["proj"]["w"], p["proj"]["b"].reshape(1, -1))


def _grid_cluster(c, bidv, validv, grid):
    n = c.shape[0]
    bid_safe = jnp.where(validv, bidv, 0)
    bmin = jax.ops.segment_min(jnp.where(validv[:, None], c, jnp.inf),
                               bid_safe, num_segments=_B)
    v = jnp.floor((c - bmin[bid_safe]) / grid).astype(jnp.int32)
    keyv = v[:, 0] * 1000000 + v[:, 1] * 1000 + v[:, 2]
    bkey = jnp.where(validv, bidv, _B)
    order = jnp.lexsort((keyv, bkey))
    sk = keyv[order]
    sb = bkey[order]
    flag = jnp.concatenate([
        jnp.zeros(1, jnp.int32),
        ((sk[1:] != sk[:-1]) | (sb[1:] != sb[:-1])).astype(jnp.int32)])
    ids = jnp.cumsum(flag)
    return jnp.zeros(n, jnp.int32).at[order].set(ids)


_STAGE_ROWS = (8192, 2816, 256, 64)
_STAGE_TT = (8, 3, 1, 1)


def _structures(coord, offset):
    n = coord.shape[0]
    bid0 = jnp.searchsorted(offset, jnp.arange(n), side="right").astype(jnp.int32)
    valid0 = jnp.ones(n, dtype=bool)
    starts0 = jnp.concatenate([jnp.zeros(1, jnp.int32), offset[:-1]])
    bcnt0 = offset - starts0
    nbr0 = _knn_pallas(coord, starts0, bcnt0, bid0, valid0, bcnt0, _PE_K)
    stages = []
    cur_c, cur_b, cur_v = coord, bid0, valid0
    for i in range(4):
        p = _STAGE_ROWS[i]
        clus = _grid_cluster(cur_c, cur_b, cur_v, _GRIDS[i])
        raw_cnt = jax.ops.segment_sum(cur_v.astype(jnp.float32), clus,
                                      num_segments=p)
        seg_valid = raw_cnt > 0
        cnt = jnp.where(seg_valid, raw_cnt, 1.0)
        pooled_c = jax.ops.segment_sum(jnp.where(cur_v[:, None], cur_c, 0.0),
                                       clus, num_segments=p) / cnt[:, None]
        seg_bid = jax.ops.segment_max(jnp.where(cur_v, cur_b, -1), clus,
                                      num_segments=p)
        seg_bid = jnp.where(seg_valid, seg_bid, 0)
        seg_bcnt = jax.ops.segment_sum(seg_valid.astype(jnp.int32), seg_bid,
                                       num_segments=_B)
        seg_starts = (jnp.cumsum(seg_bcnt) - seg_bcnt).astype(jnp.int32)
        nbr = _knn_pallas(pooled_c, seg_starts, seg_bcnt, seg_bid, seg_valid,
                          seg_bcnt, _ENC_K[i], ttiles=_STAGE_TT[i])
        stages.append((clus, p, cnt, nbr, seg_valid))
        cur_c, cur_b, cur_v = pooled_c, seg_bid, seg_valid
    bcnt = jax.ops.segment_sum(cur_v.astype(jnp.float32), cur_b,
                               num_segments=_B)
    return nbr0, stages, cur_b, bcnt


def kernel(coord, feat, offset, params):
    nbr0, stages, bid, bcnt = _structures(coord, offset)
    x = coord
    f = _plinear(feat, params["embed"]["w"], params["embed"]["b"])
    f = _gva(params["pe_block"], f, x, nbr0, _PE_G)
    for i in range(4):
        f = _plinear(f, params["down"][i]["w"], params["down"][i]["b"],
                     relu=True)
        clus, nseg, cnt, nbr, seg_valid = stages[i]
        px = jax.ops.segment_sum(x, clus, num_segments=nseg) / cnt[:, None]
        pf = jax.ops.segment_max(f, clus, num_segments=nseg)
        f = _gva(params["blocks"][i], pf, px, nbr, _ENC_G[i])
        x = px
    f = jnp.where(stages[-1][4][:, None], f, 0.0)
    pooled = jax.ops.segment_sum(f, bid, num_segments=bcnt.shape[0]) / bcnt[:, None]
    h = _plinear(pooled, params["head1"]["w"], params["head1"]["b"], relu=True)
    h = _plinear(h, params["head2"]["w"], params["head2"]["b"], relu=True)
    return _plinear(h, params["head3"]["w"], params["head3"]["b"])


# speedup vs baseline: 17.6022x; 1.7788x over previous
import functools

import jax
import jax.numpy as jnp
from jax.experimental import pallas as pl
from jax.experimental.pallas import tpu as pltpu

_B = 8
_PE_G = 6
_PE_K = 8
_ENC_CH = (96, 192, 384, 512)
_ENC_G = (12, 24, 48, 64)
_ENC_K = (16, 16, 16, 16)
_GRIDS = (0.06, 0.15, 0.375, 0.9375)
_SZ = (1024, 384, 32, 8)


def _linear_kern(x_ref, w_ref, b_ref, o_ref, *, relu):
    y = jnp.dot(x_ref[...], w_ref[...], preferred_element_type=jnp.float32)
    y = y + b_ref[...]
    if relu:
        y = jnp.maximum(y, 0.0)
    o_ref[...] = y


def _plinear(x, w, b, relu=False, tile=512):
    n, din = x.shape
    dout = w.shape[1]
    tile = min(tile, n)
    while n % tile:
        tile //= 2
    return pl.pallas_call(
        functools.partial(_linear_kern, relu=relu),
        grid=(n // tile,),
        in_specs=[
            pl.BlockSpec((tile, din), lambda i: (i, 0)),
            pl.BlockSpec((din, dout), lambda i: (0, 0)),
            pl.BlockSpec((1, dout), lambda i: (0, 0)),
        ],
        out_specs=pl.BlockSpec((tile, dout), lambda i: (i, 0)),
        out_shape=jax.ShapeDtypeStruct((n, dout), jnp.float32),
    )(x, w, b.reshape(1, -1))


def _knn_kern(cnt_ref, c_ref, caT_ref, o_ref, *, K, SZ):
    b = pl.program_id(0)
    cnt = cnt_ref[b]
    rows = c_ref[...]
    RT = rows.shape[0]
    cx = caT_ref[0, 0:1, :]
    cy = caT_ref[0, 1:2, :]
    cz = caT_ref[0, 2:3, :]
    dx = rows[:, 0:1] - cx
    dy = rows[:, 1:2] - cy
    dz = rows[:, 2:3] - cz
    d = (dx * dx + dy * dy) + dz * dz
    jl = jax.lax.broadcasted_iota(jnp.int32, (RT, SZ), 1)
    d = jnp.where(jl < cnt, d, jnp.inf)
    cols = []
    for _ in range(K):
        m = jnp.min(d, axis=1, keepdims=True)
        ismin = d == m
        fidx = jnp.min(jnp.where(ismin, jl, SZ), axis=1, keepdims=True)
        cols.append(fidx)
        d = jnp.where(jl == fidx, jnp.inf, d)
    out = jnp.concatenate(cols, axis=1).astype(jnp.int32) + b * SZ
    o_ref[...] = out


def _knn_pallas(c, cnts, bidv, validv, bcount, k, sz):
    m = c.shape[0]
    rt = min(128, sz)
    tt = sz // rt
    caT = c.reshape(_B, sz, 3).transpose(0, 2, 1)
    nb = pl.pallas_call(
        functools.partial(_knn_kern, K=k, SZ=sz),
        grid=(_B, tt),
        in_specs=[
            pl.BlockSpec(memory_space=pltpu.MemorySpace.SMEM),
            pl.BlockSpec((rt, 3), lambda b, t: (b * tt + t, 0)),
            pl.BlockSpec((1, 3, sz), lambda b, t: (b, 0, 0)),
        ],
        out_specs=pl.BlockSpec((rt, k), lambda b, t: (b * tt + t, 0)),
        out_shape=jax.ShapeDtypeStruct((m, k), jnp.int32),
    )(cnts, c, caT)
    vcnt = bcount[jnp.where(validv, bidv, 0)]
    col = jnp.arange(k)[None, :]
    last = jnp.take_along_axis(nb, jnp.clip(vcnt - 1, 0, k - 1)[:, None], axis=1)
    return jnp.where(col < vcnt[:, None], nb, last)


def _gva_kern(q_ref, feat_ref, co_ref, nbr_ref, ktab_ref, vtab_ref, cw_ref,
              pe1_ref, pe1b_ref, pe2_ref, pe2b_ref,
              we1_ref, we1b_ref, we2_ref, we2b_ref,
              gmat_ref, proj_ref, projb_ref, o_ref, *, K, G, SZ):
    TN, ch = q_ref.shape
    TNK = TN * K
    nl = nbr_ref[...]
    oh = (jax.lax.broadcasted_iota(jnp.int32, (TNK, SZ), 1) == nl)
    oh = oh.astype(jnp.float32)
    kn = jnp.dot(oh, ktab_ref[...], preferred_element_type=jnp.float32)
    vn = jnp.dot(oh, vtab_ref[...], preferred_element_type=jnp.float32)
    cn = jnp.dot(oh, cw_ref[...], preferred_element_type=jnp.float32)
    pos = (cn.reshape(TN, K, 3) - co_ref[...].reshape(TN, 1, 3)).reshape(TNK, 3)
    h = jnp.dot(pos, pe1_ref[...], preferred_element_type=jnp.float32)
    h = jnp.maximum(h + pe1b_ref[...], 0.0)
    pe = jnp.dot(h, pe2_ref[...], preferred_element_type=jnp.float32)
    pe = pe + pe2b_ref[...]
    rel = (q_ref[...].reshape(TN, 1, ch)
           - kn.reshape(TN, K, ch)
           + pe.reshape(TN, K, ch)).reshape(TNK, ch)
    t = jnp.dot(rel, we1_ref[...], preferred_element_type=jnp.float32)
    t = jnp.maximum(t + we1b_ref[...], 0.0)
    w = jnp.dot(t, we2_ref[...], preferred_element_type=jnp.float32)
    w = (w + we2b_ref[...]).reshape(TN, K, G)
    w = w - jnp.max(w, axis=1, keepdims=True)
    e = jnp.exp(w)
    w = e / jnp.sum(e, axis=1, keepdims=True)
    wfull = jnp.dot(w.reshape(TNK, G), gmat_ref[...],
                    preferred_element_type=jnp.float32)
    val = vn + pe
    out = jnp.sum((wfull * val).reshape(TN, K, ch), axis=1)
    y = jnp.dot(out, proj_ref[...], preferred_element_type=jnp.float32)
    y = y + projb_ref[...]
    o_ref[...] = feat_ref[...] + jnp.maximum(y, 0.0)


def _gva(p, feat, coord, nbr, g, sz):
    n, ch = feat.shape
    k = nbr.shape[1]
    wqkv = jnp.concatenate([p["q"]["w"], p["k"]["w"], p["v"]["w"]], axis=1)
    bqkv = jnp.concatenate([p["q"]["b"], p["k"]["b"], p["v"]["b"]])
    qkv = _plinear(feat, wqkv, bqkv)
    q = qkv[:, :ch]
    ktab = qkv[:, ch:2 * ch]
    vtab = qkv[:, 2 * ch:]
    nloc = (nbr - (jnp.arange(n, dtype=jnp.int32)[:, None] // sz) * sz
            ).reshape(n * k, 1)
    gmat = (jnp.arange(ch)[None, :] // (ch // g)
            == jnp.arange(g)[:, None]).astype(jnp.float32)
    TN = min(128, sz)
    tpb = sz // TN
    row = lambda r, c: pl.BlockSpec((r, c), lambda i: (i, 0))
    win = lambda r, c: pl.BlockSpec((r, c), lambda i: (i // tpb, 0))
    full = lambda r, c: pl.BlockSpec((r, c), lambda i: (0, 0))
    return pl.pallas_call(
        functools.partial(_gva_kern, K=k, G=g, SZ=sz),
        grid=(n // TN,),
        in_specs=[
            row(TN, ch), row(TN, ch), row(TN, 3), row(TN * k, 1),
            win(sz, ch), win(sz, ch), win(sz, 3),
            full(3, ch), full(1, ch), full(ch, ch), full(1, ch),
            full(ch, ch), full(1, ch), full(ch, g), full(1, g),
            full(g, ch), full(ch, ch), full(1, ch),
        ],
        out_specs=row(TN, ch),
        out_shape=jax.ShapeDtypeStruct((n, ch), jnp.float32),
    )(q, feat, coord, nloc, ktab, vtab, coord,
      p["pe1"]["w"], p["pe1"]["b"].reshape(1, -1),
      p["pe2"]["w"], p["pe2"]["b"].reshape(1, -1),
      p["we1"]["w"], p["we1"]["b"].reshape(1, -1),
      p["we2"]["w"], p["we2"]["b"].reshape(1, -1),
      gmat, p["proj"]["w"], p["proj"]["b"].reshape(1, -1))


def _grid_cluster(c, bidv, validv, grid):
    n = c.shape[0]
    bid_safe = jnp.where(validv, bidv, 0)
    bmin = jax.ops.segment_min(jnp.where(validv[:, None], c, jnp.inf),
                               bid_safe, num_segments=_B)
    v = jnp.floor((c - bmin[bid_safe]) / grid).astype(jnp.int32)
    keyv = v[:, 0] * 1000000 + v[:, 1] * 1000 + v[:, 2]
    bkey = jnp.where(validv, bidv, _B)
    order = jnp.lexsort((keyv, bkey))
    sk = keyv[order]
    sb = bkey[order]
    flag = jnp.concatenate([
        jnp.zeros(1, jnp.int32),
        ((sk[1:] != sk[:-1]) | (sb[1:] != sb[:-1])).astype(jnp.int32)])
    ids = jnp.cumsum(flag)
    return jnp.zeros(n, jnp.int32).at[order].set(ids)


def _structures(coord, offset):
    n = coord.shape[0]
    bid0 = jnp.searchsorted(offset, jnp.arange(n), side="right").astype(jnp.int32)
    valid0 = jnp.ones(n, dtype=bool)
    starts0 = jnp.concatenate([jnp.zeros(1, jnp.int32), offset[:-1]])
    bcnt0 = offset - starts0
    nbr0 = _knn_pallas(coord, bcnt0, bid0, valid0, bcnt0, _PE_K, _SZ[0])
    stages = []
    cur_c, cur_b, cur_v = coord, bid0, valid0
    for i in range(4):
        sz = _SZ[i]
        m = _B * sz
        clus_g = _grid_cluster(cur_c, cur_b, cur_v, _GRIDS[i])
        bid_safe = jnp.where(cur_v, cur_b, 0)
        bfirst = jax.ops.segment_min(
            jnp.where(cur_v, clus_g, jnp.int32(2 ** 30)), bid_safe,
            num_segments=_B)
        clus = jnp.where(cur_v, cur_b * sz + clus_g - bfirst[bid_safe], m)
        raw_cnt = jax.ops.segment_sum(cur_v.astype(jnp.float32), clus,
                                      num_segments=m)
        seg_valid = raw_cnt > 0
        cnt = jnp.where(seg_valid, raw_cnt, 1.0)
        pooled_c = jax.ops.segment_sum(jnp.where(cur_v[:, None], cur_c, 0.0),
                                       clus, num_segments=m) / cnt[:, None]
        seg_bid = jax.ops.segment_max(jnp.where(cur_v, cur_b, -1), clus,
                                      num_segments=m)
        seg_bid = jnp.where(seg_valid, seg_bid, 0)
        seg_bcnt = jax.ops.segment_sum(seg_valid.astype(jnp.int32), seg_bid,
                                       num_segments=_B)
        nbr = _knn_pallas(pooled_c, seg_bcnt, seg_bid, seg_valid, seg_bcnt,
                          _ENC_K[i], sz)
        stages.append((clus, m, cnt, nbr, seg_valid))
        cur_c, cur_b, cur_v = pooled_c, seg_bid, seg_valid
    bcnt = jax.ops.segment_sum(cur_v.astype(jnp.float32), cur_b,
                               num_segments=_B)
    return nbr0, stages, cur_b, bcnt


def kernel(coord, feat, offset, params):
    nbr0, stages, bid, bcnt = _structures(coord, offset)
    x = coord
    f = _plinear(feat, params["embed"]["w"], params["embed"]["b"])
    f = _gva(params["pe_block"], f, x, nbr0, _PE_G, _SZ[0])
    for i in range(4):
        f = _plinear(f, params["down"][i]["w"], params["down"][i]["b"],
                     relu=True)
        clus, nseg, cnt, nbr, seg_valid = stages[i]
        px = jax.ops.segment_sum(x, clus, num_segments=nseg) / cnt[:, None]
        pf = jax.ops.segment_max(f, clus, num_segments=nseg)
        pf = jnp.where(seg_valid[:, None], pf, 0.0)
        f = _gva(params["blocks"][i], pf, px, nbr, _ENC_G[i], _SZ[i])
        x = px
    f = jnp.where(stages[-1][4][:, None], f, 0.0)
    pooled = jax.ops.segment_sum(f, bid, num_segments=bcnt.shape[0]) / bcnt[:, None]
    h = _plinear(pooled, params["head1"]["w"], params["head1"]["b"], relu=True)
    h = _plinear(h, params["head2"]["w"], params["head2"]["b"], relu=True)
    return _plinear(h, params["head3"]["w"], params["head3"]["b"])


# speedup vs baseline: 21.9691x; 1.2481x over previous
import functools

import jax
import jax.numpy as jnp
from jax.experimental import pallas as pl
from jax.experimental.pallas import tpu as pltpu

_B = 8
_PE_G = 6
_PE_K = 8
_ENC_CH = (96, 192, 384, 512)
_ENC_G = (12, 24, 48, 64)
_ENC_K = (16, 16, 16, 16)
_GRIDS = (0.06, 0.15, 0.375, 0.9375)
_SZ = (1024, 384, 32, 8)


def _linear_kern(x_ref, w_ref, b_ref, o_ref, *, relu):
    y = jnp.dot(x_ref[...], w_ref[...], preferred_element_type=jnp.float32)
    y = y + b_ref[...]
    if relu:
        y = jnp.maximum(y, 0.0)
    o_ref[...] = y


def _plinear(x, w, b, relu=False, tile=512):
    n, din = x.shape
    dout = w.shape[1]
    tile = min(tile, n)
    while n % tile:
        tile //= 2
    return pl.pallas_call(
        functools.partial(_linear_kern, relu=relu),
        grid=(n // tile,),
        in_specs=[
            pl.BlockSpec((tile, din), lambda i: (i, 0)),
            pl.BlockSpec((din, dout), lambda i: (0, 0)),
            pl.BlockSpec((1, dout), lambda i: (0, 0)),
        ],
        out_specs=pl.BlockSpec((tile, dout), lambda i: (i, 0)),
        out_shape=jax.ShapeDtypeStruct((n, dout), jnp.float32),
    )(x, w, b.reshape(1, -1))


def _knn_kern(cnt_ref, c_ref, caT_ref, o_ref, *, K, SZ):
    b = pl.program_id(0)
    cnt = cnt_ref[b]
    rows = c_ref[...]
    RT = rows.shape[0]
    cx = caT_ref[0, 0:1, :]
    cy = caT_ref[0, 1:2, :]
    cz = caT_ref[0, 2:3, :]
    dx = rows[:, 0:1] - cx
    dy = rows[:, 1:2] - cy
    dz = rows[:, 2:3] - cz
    d = (dx * dx + dy * dy) + dz * dz
    jl = jax.lax.broadcasted_iota(jnp.int32, (RT, SZ), 1)
    d = jnp.where(jl < cnt, d, jnp.inf)
    cols = []
    for _ in range(K):
        m = jnp.min(d, axis=1, keepdims=True)
        ismin = d == m
        fidx = jnp.min(jnp.where(ismin, jl, SZ), axis=1, keepdims=True)
        cols.append(fidx)
        d = jnp.where(jl == fidx, jnp.inf, d)
    out = jnp.concatenate(cols, axis=1).astype(jnp.int32) + b * SZ
    o_ref[...] = out


def _knn_pallas(c, cnts, bidv, validv, bcount, k, sz):
    m = c.shape[0]
    rt = min(128, sz)
    tt = sz // rt
    caT = c.reshape(_B, sz, 3).transpose(0, 2, 1)
    nb = pl.pallas_call(
        functools.partial(_knn_kern, K=k, SZ=sz),
        grid=(_B, tt),
        in_specs=[
            pl.BlockSpec(memory_space=pltpu.MemorySpace.SMEM),
            pl.BlockSpec((rt, 3), lambda b, t: (b * tt + t, 0)),
            pl.BlockSpec((1, 3, sz), lambda b, t: (b, 0, 0)),
        ],
        out_specs=pl.BlockSpec((rt, k), lambda b, t: (b * tt + t, 0)),
        out_shape=jax.ShapeDtypeStruct((m, k), jnp.int32),
    )(cnts, c, caT)
    vcnt = bcount[jnp.where(validv, bidv, 0)]
    col = jnp.arange(k)[None, :]
    last = jnp.take_along_axis(nb, jnp.clip(vcnt - 1, 0, k - 1)[:, None], axis=1)
    return jnp.where(col < vcnt[:, None], nb, last)


def _gva_kern(q_ref, feat_ref, co_ref, nbr_ref, ktab_ref, vtab_ref, cw_ref,
              pe1_ref, pe1b_ref, pe2_ref, pe2b_ref,
              we1_ref, we1b_ref, we2_ref, we2b_ref,
              gmat_ref, proj_ref, projb_ref, o_ref, *, K, G, SZ):
    TN, ch = q_ref.shape
    TNK = TN * K
    nl = nbr_ref[...]
    oh = (jax.lax.broadcasted_iota(jnp.int32, (TNK, SZ), 1) == nl)
    oh = oh.astype(jnp.float32)
    kn = jnp.dot(oh, ktab_ref[...], preferred_element_type=jnp.float32)
    vn = jnp.dot(oh, vtab_ref[...], preferred_element_type=jnp.float32)
    cn = jnp.dot(oh, cw_ref[...], preferred_element_type=jnp.float32)
    pos = (cn.reshape(TN, K, 3) - co_ref[...].reshape(TN, 1, 3)).reshape(TNK, 3)
    h = jnp.dot(pos, pe1_ref[...], preferred_element_type=jnp.float32)
    h = jnp.maximum(h + pe1b_ref[...], 0.0)
    pe = jnp.dot(h, pe2_ref[...], preferred_element_type=jnp.float32)
    pe = pe + pe2b_ref[...]
    rel = (q_ref[...].reshape(TN, 1, ch)
           - kn.reshape(TN, K, ch)
           + pe.reshape(TN, K, ch)).reshape(TNK, ch)
    t = jnp.dot(rel, we1_ref[...], preferred_element_type=jnp.float32)
    t = jnp.maximum(t + we1b_ref[...], 0.0)
    w = jnp.dot(t, we2_ref[...], preferred_element_type=jnp.float32)
    w = (w + we2b_ref[...]).reshape(TN, K, G)
    w = w - jnp.max(w, axis=1, keepdims=True)
    e = jnp.exp(w)
    w = e / jnp.sum(e, axis=1, keepdims=True)
    wfull = jnp.dot(w.reshape(TNK, G), gmat_ref[...],
                    preferred_element_type=jnp.float32)
    val = vn + pe
    out = jnp.sum((wfull * val).reshape(TN, K, ch), axis=1)
    y = jnp.dot(out, proj_ref[...], preferred_element_type=jnp.float32)
    y = y + projb_ref[...]
    o_ref[...] = feat_ref[...] + jnp.maximum(y, 0.0)


def _gva(p, feat, coord, nbr, g, sz):
    n, ch = feat.shape
    k = nbr.shape[1]
    wqkv = jnp.concatenate([p["q"]["w"], p["k"]["w"], p["v"]["w"]], axis=1)
    bqkv = jnp.concatenate([p["q"]["b"], p["k"]["b"], p["v"]["b"]])
    qkv = _plinear(feat, wqkv, bqkv)
    q = qkv[:, :ch]
    ktab = qkv[:, ch:2 * ch]
    vtab = qkv[:, 2 * ch:]
    nloc = (nbr - (jnp.arange(n, dtype=jnp.int32)[:, None] // sz) * sz
            ).reshape(n * k, 1)
    gmat = (jnp.arange(ch)[None, :] // (ch // g)
            == jnp.arange(g)[:, None]).astype(jnp.float32)
    TN = min(128, sz)
    tpb = sz // TN
    row = lambda r, c: pl.BlockSpec((r, c), lambda i: (i, 0))
    win = lambda r, c: pl.BlockSpec((r, c), lambda i: (i // tpb, 0))
    full = lambda r, c: pl.BlockSpec((r, c), lambda i: (0, 0))
    return pl.pallas_call(
        functools.partial(_gva_kern, K=k, G=g, SZ=sz),
        grid=(n // TN,),
        in_specs=[
            row(TN, ch), row(TN, ch), row(TN, 3), row(TN * k, 1),
            win(sz, ch), win(sz, ch), win(sz, 3),
            full(3, ch), full(1, ch), full(ch, ch), full(1, ch),
            full(ch, ch), full(1, ch), full(ch, g), full(1, g),
            full(g, ch), full(ch, ch), full(1, ch),
        ],
        out_specs=row(TN, ch),
        out_shape=jax.ShapeDtypeStruct((n, ch), jnp.float32),
    )(q, feat, coord, nloc, ktab, vtab, coord,
      p["pe1"]["w"], p["pe1"]["b"].reshape(1, -1),
      p["pe2"]["w"], p["pe2"]["b"].reshape(1, -1),
      p["we1"]["w"], p["we1"]["b"].reshape(1, -1),
      p["we2"]["w"], p["we2"]["b"].reshape(1, -1),
      gmat, p["proj"]["w"], p["proj"]["b"].reshape(1, -1))


def _grid_cluster(c, bidv, validv, grid):
    n = c.shape[0]
    bid_safe = jnp.where(validv, bidv, 0)
    bmin = jax.ops.segment_min(jnp.where(validv[:, None], c, jnp.inf),
                               bid_safe, num_segments=_B)
    v = jnp.floor((c - bmin[bid_safe]) / grid).astype(jnp.int32)
    keyv = v[:, 0] * 1000000 + v[:, 1] * 1000 + v[:, 2]
    bkey = jnp.where(validv, bidv, _B)
    order = jnp.lexsort((keyv, bkey))
    sk = keyv[order]
    sb = bkey[order]
    flag = jnp.concatenate([
        jnp.zeros(1, jnp.int32),
        ((sk[1:] != sk[:-1]) | (sb[1:] != sb[:-1])).astype(jnp.int32)])
    ids = jnp.cumsum(flag)
    return jnp.zeros(n, jnp.int32).at[order].set(ids)


def _cluskey_kern(cnt_ref, c_ref, caT_ref, key_ref, rep_ref, *, SZ, RT, GRID):
    b = pl.program_id(0)
    t = pl.program_id(1)
    cnt = cnt_ref[b]
    rows = c_ref[...]
    cx = caT_ref[0, 0:1, :]
    cy = caT_ref[0, 1:2, :]
    cz = caT_ref[0, 2:3, :]
    lane = jax.lax.broadcasted_iota(jnp.int32, (1, SZ), 1)
    lv = lane < cnt
    bmx = jnp.min(jnp.where(lv, cx, jnp.inf), axis=1, keepdims=True)
    bmy = jnp.min(jnp.where(lv, cy, jnp.inf), axis=1, keepdims=True)
    bmz = jnp.min(jnp.where(lv, cz, jnp.inf), axis=1, keepdims=True)
    kw = (jnp.floor((cx - bmx) / GRID).astype(jnp.int32) * 1000000
          + jnp.floor((cy - bmy) / GRID).astype(jnp.int32) * 1000
          + jnp.floor((cz - bmz) / GRID).astype(jnp.int32))
    kr = (jnp.floor((rows[:, 0:1] - bmx) / GRID).astype(jnp.int32) * 1000000
          + jnp.floor((rows[:, 1:2] - bmy) / GRID).astype(jnp.int32) * 1000
          + jnp.floor((rows[:, 2:3] - bmz) / GRID).astype(jnp.int32))
    rloc = t * RT + jax.lax.broadcasted_iota(jnp.int32, (RT, 1), 0)
    dup = lv & (lane < rloc) & (kw == kr)
    rep = jnp.logical_not(jnp.any(dup, axis=1, keepdims=True))
    rep = rep & (rloc < cnt)
    key_ref[...] = kr
    rep_ref[...] = rep.astype(jnp.int32)


def _clusrank_kern(cnt_ref, key_ref, keyT_ref, repT_ref, rank_ref, *, SZ):
    b = pl.program_id(0)
    cnt = cnt_ref[b]
    kr = key_ref[...]
    kw = keyT_ref[0, 0:1, :]
    rw = repT_ref[0, 0:1, :]
    lane = jax.lax.broadcasted_iota(jnp.int32, (1, SZ), 1)
    cmp = (lane < cnt) & (rw > 0) & (kw < kr)
    rank_ref[...] = jnp.sum(cmp.astype(jnp.int32), axis=1, keepdims=True)


def _cluster_aligned(cur_c, cnt_in, sz_in, sz_out, grid):
    mm = cur_c.shape[0]
    rt = min(128, sz_in)
    tt = sz_in // rt
    caT = cur_c.reshape(_B, sz_in, 3).transpose(0, 2, 1)
    key, rep = pl.pallas_call(
        functools.partial(_cluskey_kern, SZ=sz_in, RT=rt, GRID=grid),
        grid=(_B, tt),
        in_specs=[
            pl.BlockSpec(memory_space=pltpu.MemorySpace.SMEM),
            pl.BlockSpec((rt, 3), lambda b, t: (b * tt + t, 0)),
            pl.BlockSpec((1, 3, sz_in), lambda b, t: (b, 0, 0)),
        ],
        out_specs=[pl.BlockSpec((rt, 1), lambda b, t: (b * tt + t, 0)),
                   pl.BlockSpec((rt, 1), lambda b, t: (b * tt + t, 0))],
        out_shape=[jax.ShapeDtypeStruct((mm, 1), jnp.int32),
                   jax.ShapeDtypeStruct((mm, 1), jnp.int32)],
    )(cnt_in, cur_c, caT)
    keyT = key.reshape(_B, 1, sz_in)
    repT = rep.reshape(_B, 1, sz_in)
    rank = pl.pallas_call(
        functools.partial(_clusrank_kern, SZ=sz_in),
        grid=(_B, tt),
        in_specs=[
            pl.BlockSpec(memory_space=pltpu.MemorySpace.SMEM),
            pl.BlockSpec((rt, 1), lambda b, t: (b * tt + t, 0)),
            pl.BlockSpec((1, 1, sz_in), lambda b, t: (b, 0, 0)),
            pl.BlockSpec((1, 1, sz_in), lambda b, t: (b, 0, 0)),
        ],
        out_specs=pl.BlockSpec((rt, 1), lambda b, t: (b * tt + t, 0)),
        out_shape=jax.ShapeDtypeStruct((mm, 1), jnp.int32),
    )(cnt_in, key, keyT, repT)
    idx = jnp.arange(mm, dtype=jnp.int32)
    bid = idx // sz_in
    validv = (idx % sz_in) < cnt_in[bid]
    clus = jnp.where(validv, bid * sz_out + rank[:, 0], _B * sz_out)
    seg_bcnt = jnp.sum(rep.reshape(_B, sz_in), axis=1, dtype=jnp.int32)
    return clus, seg_bcnt


def _structures(coord, offset):
    n = coord.shape[0]
    bid0 = jnp.searchsorted(offset, jnp.arange(n), side="right").astype(jnp.int32)
    valid0 = jnp.ones(n, dtype=bool)
    starts0 = jnp.concatenate([jnp.zeros(1, jnp.int32), offset[:-1]])
    bcnt0 = offset - starts0
    nbr0 = _knn_pallas(coord, bcnt0, bid0, valid0, bcnt0, _PE_K, _SZ[0])
    stages = []
    cur_c, cur_v, cnt_in, sz_in = coord, valid0, bcnt0, _SZ[0]
    for i in range(4):
        sz = _SZ[i]
        m = _B * sz
        clus, seg_bcnt = _cluster_aligned(cur_c, cnt_in, sz_in, sz, _GRIDS[i])
        idx = jnp.arange(m, dtype=jnp.int32)
        seg_bid = idx // sz
        seg_valid = (idx % sz) < seg_bcnt[seg_bid]
        raw_cnt = jax.ops.segment_sum(cur_v.astype(jnp.float32), clus,
                                      num_segments=m)
        cnt = jnp.where(seg_valid, raw_cnt, 1.0)
        pooled_c = jax.ops.segment_sum(jnp.where(cur_v[:, None], cur_c, 0.0),
                                       clus, num_segments=m) / cnt[:, None]
        seg_bid = jnp.where(seg_valid, seg_bid, 0)
        nbr = _knn_pallas(pooled_c, seg_bcnt, seg_bid, seg_valid, seg_bcnt,
                          _ENC_K[i], sz)
        stages.append((clus, m, cnt, nbr, seg_valid))
        cur_c, cur_v, cnt_in, sz_in = pooled_c, seg_valid, seg_bcnt, sz
    bid = jnp.where(cur_v, jnp.arange(_B * _SZ[3], dtype=jnp.int32) // _SZ[3], 0)
    bcnt = cnt_in.astype(jnp.float32)
    return nbr0, stages, bid, bcnt


def kernel(coord, feat, offset, params):
    nbr0, stages, bid, bcnt = _structures(coord, offset)
    x = coord
    f = _plinear(feat, params["embed"]["w"], params["embed"]["b"])
    f = _gva(params["pe_block"], f, x, nbr0, _PE_G, _SZ[0])
    for i in range(4):
        f = _plinear(f, params["down"][i]["w"], params["down"][i]["b"],
                     relu=True)
        clus, nseg, cnt, nbr, seg_valid = stages[i]
        px = jax.ops.segment_sum(x, clus, num_segments=nseg) / cnt[:, None]
        pf = jax.ops.segment_max(f, clus, num_segments=nseg)
        pf = jnp.where(seg_valid[:, None], pf, 0.0)
        f = _gva(params["blocks"][i], pf, px, nbr, _ENC_G[i], _SZ[i])
        x = px
    f = jnp.where(stages[-1][4][:, None], f, 0.0)
    pooled = jax.ops.segment_sum(f, bid, num_segments=bcnt.shape[0]) / bcnt[:, None]
    h = _plinear(pooled, params["head1"]["w"], params["head1"]["b"], relu=True)
    h = _plinear(h, params["head2"]["w"], params["head2"]["b"], relu=True)
    return _plinear(h, params["head3"]["w"], params["head3"]["b"])


# speedup vs baseline: 22.6024x; 1.0288x over previous
import functools

import jax
import jax.numpy as jnp
from jax.experimental import pallas as pl
from jax.experimental.pallas import tpu as pltpu

_B = 8
_PE_G = 6
_PE_K = 8
_ENC_CH = (96, 192, 384, 512)
_ENC_G = (12, 24, 48, 64)
_ENC_K = (16, 16, 16, 16)
_GRIDS = (0.06, 0.15, 0.375, 0.9375)
_SZ = (1024, 384, 32, 8)


def _linear_kern(x_ref, w_ref, b_ref, o_ref, *, relu):
    y = jnp.dot(x_ref[...], w_ref[...], preferred_element_type=jnp.float32)
    y = y + b_ref[...]
    if relu:
        y = jnp.maximum(y, 0.0)
    o_ref[...] = y


def _plinear(x, w, b, relu=False, tile=512):
    n, din = x.shape
    dout = w.shape[1]
    tile = min(tile, n)
    while n % tile:
        tile //= 2
    return pl.pallas_call(
        functools.partial(_linear_kern, relu=relu),
        grid=(n // tile,),
        in_specs=[
            pl.BlockSpec((tile, din), lambda i: (i, 0)),
            pl.BlockSpec((din, dout), lambda i: (0, 0)),
            pl.BlockSpec((1, dout), lambda i: (0, 0)),
        ],
        out_specs=pl.BlockSpec((tile, dout), lambda i: (i, 0)),
        out_shape=jax.ShapeDtypeStruct((n, dout), jnp.float32),
    )(x, w, b.reshape(1, -1))


def _knn_kern(cnt_ref, c_ref, caT_ref, o_ref, *, K, SZ):
    b = pl.program_id(0)
    cnt = cnt_ref[b]
    rows = c_ref[...]
    RT = rows.shape[0]
    cx = caT_ref[0, 0:1, :]
    cy = caT_ref[0, 1:2, :]
    cz = caT_ref[0, 2:3, :]
    dx = rows[:, 0:1] - cx
    dy = rows[:, 1:2] - cy
    dz = rows[:, 2:3] - cz
    d = (dx * dx + dy * dy) + dz * dz
    jl = jax.lax.broadcasted_iota(jnp.int32, (RT, SZ), 1)
    d = jnp.where(jl < cnt, d, jnp.inf)
    cols = []
    for _ in range(K):
        m = jnp.min(d, axis=1, keepdims=True)
        ismin = d == m
        fidx = jnp.min(jnp.where(ismin, jl, SZ), axis=1, keepdims=True)
        cols.append(fidx)
        d = jnp.where(jl == fidx, jnp.inf, d)
    out = jnp.concatenate(cols, axis=1).astype(jnp.int32) + b * SZ
    o_ref[...] = out


def _knn_pallas(c, cnts, bidv, validv, bcount, k, sz):
    m = c.shape[0]
    rt = min(128, sz)
    tt = sz // rt
    caT = c.reshape(_B, sz, 3).transpose(0, 2, 1)
    nb = pl.pallas_call(
        functools.partial(_knn_kern, K=k, SZ=sz),
        grid=(_B, tt),
        in_specs=[
            pl.BlockSpec(memory_space=pltpu.MemorySpace.SMEM),
            pl.BlockSpec((rt, 3), lambda b, t: (b * tt + t, 0)),
            pl.BlockSpec((1, 3, sz), lambda b, t: (b, 0, 0)),
        ],
        out_specs=pl.BlockSpec((rt, k), lambda b, t: (b * tt + t, 0)),
        out_shape=jax.ShapeDtypeStruct((m, k), jnp.int32),
    )(cnts, c, caT)
    vcnt = bcount[jnp.where(validv, bidv, 0)]
    col = jnp.arange(k)[None, :]
    last = jnp.take_along_axis(nb, jnp.clip(vcnt - 1, 0, k - 1)[:, None], axis=1)
    return jnp.where(col < vcnt[:, None], nb, last)


def _gva_kern(q_ref, feat_ref, co_ref, nbr_ref, ktab_ref, vtab_ref, cw_ref,
              pe1_ref, pe1b_ref, pe2_ref, pe2b_ref,
              we1_ref, we1b_ref, we2_ref, we2b_ref,
              gmat_ref, proj_ref, projb_ref, o_ref, *, K, G, SZ):
    TN, ch = q_ref.shape
    TNK = TN * K
    nl = nbr_ref[...]
    oh = (jax.lax.broadcasted_iota(jnp.int32, (TNK, SZ), 1) == nl)
    oh = oh.astype(jnp.float32)
    kn = jnp.dot(oh, ktab_ref[...], preferred_element_type=jnp.float32)
    vn = jnp.dot(oh, vtab_ref[...], preferred_element_type=jnp.float32)
    cn = jnp.dot(oh, cw_ref[...], preferred_element_type=jnp.float32)
    pos = (cn.reshape(TN, K, 3) - co_ref[...].reshape(TN, 1, 3)).reshape(TNK, 3)
    h = jnp.dot(pos, pe1_ref[...], preferred_element_type=jnp.float32)
    h = jnp.maximum(h + pe1b_ref[...], 0.0)
    pe = jnp.dot(h, pe2_ref[...], preferred_element_type=jnp.float32)
    pe = pe + pe2b_ref[...]
    rel = (q_ref[...].reshape(TN, 1, ch)
           - kn.reshape(TN, K, ch)
           + pe.reshape(TN, K, ch)).reshape(TNK, ch)
    t = jnp.dot(rel, we1_ref[...], preferred_element_type=jnp.float32)
    t = jnp.maximum(t + we1b_ref[...], 0.0)
    w = jnp.dot(t, we2_ref[...], preferred_element_type=jnp.float32)
    w = (w + we2b_ref[...]).reshape(TN, K, G)
    w = w - jnp.max(w, axis=1, keepdims=True)
    e = jnp.exp(w)
    w = e / jnp.sum(e, axis=1, keepdims=True)
    wfull = jnp.dot(w.reshape(TNK, G), gmat_ref[...],
                    preferred_element_type=jnp.float32)
    val = vn + pe
    out = jnp.sum((wfull * val).reshape(TN, K, ch), axis=1)
    y = jnp.dot(out, proj_ref[...], preferred_element_type=jnp.float32)
    y = y + projb_ref[...]
    o_ref[...] = feat_ref[...] + jnp.maximum(y, 0.0)


def _gva(p, feat, coord, nbr, g, sz):
    n, ch = feat.shape
    k = nbr.shape[1]
    wqkv = jnp.concatenate([p["q"]["w"], p["k"]["w"], p["v"]["w"]], axis=1)
    bqkv = jnp.concatenate([p["q"]["b"], p["k"]["b"], p["v"]["b"]])
    qkv = _plinear(feat, wqkv, bqkv)
    q = qkv[:, :ch]
    ktab = qkv[:, ch:2 * ch]
    vtab = qkv[:, 2 * ch:]
    nloc = (nbr - (jnp.arange(n, dtype=jnp.int32)[:, None] // sz) * sz
            ).reshape(n * k, 1)
    gmat = (jnp.arange(ch)[None, :] // (ch // g)
            == jnp.arange(g)[:, None]).astype(jnp.float32)
    TN = min(128, sz)
    tpb = sz // TN
    row = lambda r, c: pl.BlockSpec((r, c), lambda i: (i, 0))
    win = lambda r, c: pl.BlockSpec((r, c), lambda i: (i // tpb, 0))
    full = lambda r, c: pl.BlockSpec((r, c), lambda i: (0, 0))
    return pl.pallas_call(
        functools.partial(_gva_kern, K=k, G=g, SZ=sz),
        grid=(n // TN,),
        in_specs=[
            row(TN, ch), row(TN, ch), row(TN, 3), row(TN * k, 1),
            win(sz, ch), win(sz, ch), win(sz, 3),
            full(3, ch), full(1, ch), full(ch, ch), full(1, ch),
            full(ch, ch), full(1, ch), full(ch, g), full(1, g),
            full(g, ch), full(ch, ch), full(1, ch),
        ],
        out_specs=row(TN, ch),
        out_shape=jax.ShapeDtypeStruct((n, ch), jnp.float32),
    )(q, feat, coord, nloc, ktab, vtab, coord,
      p["pe1"]["w"], p["pe1"]["b"].reshape(1, -1),
      p["pe2"]["w"], p["pe2"]["b"].reshape(1, -1),
      p["we1"]["w"], p["we1"]["b"].reshape(1, -1),
      p["we2"]["w"], p["we2"]["b"].reshape(1, -1),
      gmat, p["proj"]["w"], p["proj"]["b"].reshape(1, -1))


def _grid_cluster(c, bidv, validv, grid):
    n = c.shape[0]
    bid_safe = jnp.where(validv, bidv, 0)
    bmin = jax.ops.segment_min(jnp.where(validv[:, None], c, jnp.inf),
                               bid_safe, num_segments=_B)
    v = jnp.floor((c - bmin[bid_safe]) / grid).astype(jnp.int32)
    keyv = v[:, 0] * 1000000 + v[:, 1] * 1000 + v[:, 2]
    bkey = jnp.where(validv, bidv, _B)
    order = jnp.lexsort((keyv, bkey))
    sk = keyv[order]
    sb = bkey[order]
    flag = jnp.concatenate([
        jnp.zeros(1, jnp.int32),
        ((sk[1:] != sk[:-1]) | (sb[1:] != sb[:-1])).astype(jnp.int32)])
    ids = jnp.cumsum(flag)
    return jnp.zeros(n, jnp.int32).at[order].set(ids)


def _cluskey_kern(cnt_ref, c_ref, caT_ref, key_ref, rep_ref, *, SZ, RT, GRID):
    b = pl.program_id(0)
    t = pl.program_id(1)
    cnt = cnt_ref[b]
    rows = c_ref[...]
    cx = caT_ref[0, 0:1, :]
    cy = caT_ref[0, 1:2, :]
    cz = caT_ref[0, 2:3, :]
    lane = jax.lax.broadcasted_iota(jnp.int32, (1, SZ), 1)
    lv = lane < cnt
    bmx = jnp.min(jnp.where(lv, cx, jnp.inf), axis=1, keepdims=True)
    bmy = jnp.min(jnp.where(lv, cy, jnp.inf), axis=1, keepdims=True)
    bmz = jnp.min(jnp.where(lv, cz, jnp.inf), axis=1, keepdims=True)
    kw = (jnp.floor((cx - bmx) / GRID).astype(jnp.int32) * 1000000
          + jnp.floor((cy - bmy) / GRID).astype(jnp.int32) * 1000
          + jnp.floor((cz - bmz) / GRID).astype(jnp.int32))
    kr = (jnp.floor((rows[:, 0:1] - bmx) / GRID).astype(jnp.int32) * 1000000
          + jnp.floor((rows[:, 1:2] - bmy) / GRID).astype(jnp.int32) * 1000
          + jnp.floor((rows[:, 2:3] - bmz) / GRID).astype(jnp.int32))
    rloc = t * RT + jax.lax.broadcasted_iota(jnp.int32, (RT, 1), 0)
    dup = lv & (lane < rloc) & (kw == kr)
    rep = jnp.logical_not(jnp.any(dup, axis=1, keepdims=True))
    rep = rep & (rloc < cnt)
    key_ref[...] = kr
    rep_ref[...] = rep.astype(jnp.int32)


def _clusrank_kern(cnt_ref, key_ref, keyT_ref, repT_ref, rank_ref, *, SZ):
    b = pl.program_id(0)
    cnt = cnt_ref[b]
    kr = key_ref[...]
    kw = keyT_ref[0, 0:1, :]
    rw = repT_ref[0, 0:1, :]
    lane = jax.lax.broadcasted_iota(jnp.int32, (1, SZ), 1)
    cmp = (lane < cnt) & (rw > 0) & (kw < kr)
    rank_ref[...] = jnp.sum(cmp.astype(jnp.int32), axis=1, keepdims=True)


def _cluster_aligned(cur_c, cnt_in, sz_in, sz_out, grid):
    mm = cur_c.shape[0]
    rt = min(128, sz_in)
    tt = sz_in // rt
    caT = cur_c.reshape(_B, sz_in, 3).transpose(0, 2, 1)
    key, rep = pl.pallas_call(
        functools.partial(_cluskey_kern, SZ=sz_in, RT=rt, GRID=grid),
        grid=(_B, tt),
        in_specs=[
            pl.BlockSpec(memory_space=pltpu.MemorySpace.SMEM),
            pl.BlockSpec((rt, 3), lambda b, t: (b * tt + t, 0)),
            pl.BlockSpec((1, 3, sz_in), lambda b, t: (b, 0, 0)),
        ],
        out_specs=[pl.BlockSpec((rt, 1), lambda b, t: (b * tt + t, 0)),
                   pl.BlockSpec((rt, 1), lambda b, t: (b * tt + t, 0))],
        out_shape=[jax.ShapeDtypeStruct((mm, 1), jnp.int32),
                   jax.ShapeDtypeStruct((mm, 1), jnp.int32)],
    )(cnt_in, cur_c, caT)
    keyT = key.reshape(_B, 1, sz_in)
    repT = rep.reshape(_B, 1, sz_in)
    rank = pl.pallas_call(
        functools.partial(_clusrank_kern, SZ=sz_in),
        grid=(_B, tt),
        in_specs=[
            pl.BlockSpec(memory_space=pltpu.MemorySpace.SMEM),
            pl.BlockSpec((rt, 1), lambda b, t: (b * tt + t, 0)),
            pl.BlockSpec((1, 1, sz_in), lambda b, t: (b, 0, 0)),
            pl.BlockSpec((1, 1, sz_in), lambda b, t: (b, 0, 0)),
        ],
        out_specs=pl.BlockSpec((rt, 1), lambda b, t: (b * tt + t, 0)),
        out_shape=jax.ShapeDtypeStruct((mm, 1), jnp.int32),
    )(cnt_in, key, keyT, repT)
    idx = jnp.arange(mm, dtype=jnp.int32)
    bid = idx // sz_in
    validv = (idx % sz_in) < cnt_in[bid]
    clus = jnp.where(validv, bid * sz_out + rank[:, 0], _B * sz_out)
    seg_bcnt = jnp.sum(rep.reshape(_B, sz_in), axis=1, dtype=jnp.int32)
    rk = rank[:, 0].reshape(_B, sz_in)
    lv = jnp.arange(sz_in)[None, :] < cnt_in[:, None]
    oh = ((rk[:, None, :] == jnp.arange(sz_out)[None, :, None])
          & lv[:, None, :]).astype(jnp.float32)
    sums = jnp.einsum('bsq,bqc->bsc', oh, cur_c.reshape(_B, sz_in, 3),
                      preferred_element_type=jnp.float32)
    szs = jnp.sum(oh, axis=2)
    seg_valid = jnp.arange(sz_out)[None, :] < seg_bcnt[:, None]
    cnt = jnp.where(seg_valid, szs, 1.0)
    pooled = (sums / cnt[..., None]).reshape(_B * sz_out, 3)
    return (clus, seg_bcnt, pooled, cnt.reshape(-1),
            seg_valid.reshape(-1))


def _structures(coord, offset):
    n = coord.shape[0]
    bid0 = jnp.searchsorted(offset, jnp.arange(n), side="right").astype(jnp.int32)
    valid0 = jnp.ones(n, dtype=bool)
    starts0 = jnp.concatenate([jnp.zeros(1, jnp.int32), offset[:-1]])
    bcnt0 = offset - starts0
    nbr0 = _knn_pallas(coord, bcnt0, bid0, valid0, bcnt0, _PE_K, _SZ[0])
    stages = []
    cur_c, cur_v, cnt_in, sz_in = coord, valid0, bcnt0, _SZ[0]
    for i in range(4):
        sz = _SZ[i]
        m = _B * sz
        clus, seg_bcnt, pooled_c, cnt, seg_valid = _cluster_aligned(
            cur_c, cnt_in, sz_in, sz, _GRIDS[i])
        seg_bid = jnp.where(seg_valid,
                            jnp.arange(m, dtype=jnp.int32) // sz, 0)
        nbr = _knn_pallas(pooled_c, seg_bcnt, seg_bid, seg_valid, seg_bcnt,
                          _ENC_K[i], sz)
        stages.append((clus, m, cnt, nbr, seg_valid, pooled_c))
        cur_c, cur_v, cnt_in, sz_in = pooled_c, seg_valid, seg_bcnt, sz
    bid = jnp.where(cur_v, jnp.arange(_B * _SZ[3], dtype=jnp.int32) // _SZ[3], 0)
    bcnt = cnt_in.astype(jnp.float32)
    return nbr0, stages, bid, bcnt


def kernel(coord, feat, offset, params):
    nbr0, stages, bid, bcnt = _structures(coord, offset)
    x = coord
    f = _plinear(feat, params["embed"]["w"], params["embed"]["b"])
    f = _gva(params["pe_block"], f, x, nbr0, _PE_G, _SZ[0])
    for i in range(4):
        f = _plinear(f, params["down"][i]["w"], params["down"][i]["b"],
                     relu=True)
        clus, nseg, cnt, nbr, seg_valid, px = stages[i]
        pf = jax.ops.segment_max(f, clus, num_segments=nseg)
        pf = jnp.where(seg_valid[:, None], pf, 0.0)
        f = _gva(params["blocks"][i], pf, px, nbr, _ENC_G[i], _SZ[i])
        x = px
    f = jnp.where(stages[-1][4][:, None], f, 0.0)
    pooled = jax.ops.segment_sum(f, bid, num_segments=bcnt.shape[0]) / bcnt[:, None]
    h = _plinear(pooled, params["head1"]["w"], params["head1"]["b"], relu=True)
    h = _plinear(h, params["head2"]["w"], params["head2"]["b"], relu=True)
    return _plinear(h, params["head3"]["w"], params["head3"]["b"])
